# Initial kernel scaffold; baseline (speedup 1.0000x reference)
import jax
import jax.numpy as jnp
from jax import lax
from jax.experimental import pallas as pl
from jax.experimental.pallas import tpu as pltpu
from jax.experimental.pallas import tpu_sc as plsc

N_USERS_K = 50000
N_ITEMS_K = 50000
N_NODES_K = N_USERS_K + N_ITEMS_K
N_EDGES_K = 1600000
EMB_K = 32

NCORES = 2
NSUB = 16
NW = NCORES * NSUB
ROWS_W = N_NODES_K // NW
ROWS_CORE = N_NODES_K // NCORES
CHUNK = 512
SUB = 128
NSUBC = CHUNK // SUB
LANES = 16


def _spmm_body(ego_hbm, cols_hbm, rows_hbm, vals_hbm, bounds_hbm, out_hbm,
               boundsv, colsv, rowsv, valsv, lrowsv, gath, acc_sh, sem):
    c = lax.axis_index("c")
    s = lax.axis_index("s")
    wid = c * NSUB + s
    base = wid * ROWS_W
    acc_lo = s * ROWS_W

    pltpu.sync_copy(bounds_hbm, boundsv)
    lo = boundsv[wid]
    hi = boundsv[wid + 1]

    zvec = jnp.zeros((LANES,), jnp.float32)

    def zero_body(i, _):
        gath[i >> 1, pl.ds((i & 1) * LANES, LANES)] = zvec
        return 0

    lax.fori_loop(0, CHUNK * 2, zero_body, 0)
    for k in range(ROWS_W // CHUNK):
        pltpu.sync_copy(gath, acc_sh.at[pl.ds(acc_lo + k * CHUNK, CHUNK)])
    rem = ROWS_W % CHUNK
    if rem:
        pltpu.sync_copy(
            gath.at[pl.ds(0, rem)],
            acc_sh.at[pl.ds(acc_lo + (ROWS_W // CHUNK) * CHUNK, rem)])

    lo_pad = (lo // CHUNK) * CHUNK
    nchunks = (hi - lo_pad + CHUNK - 1) // CHUNK
    iota = lax.iota(jnp.int32, LANES)

    def chunk_body(ci, _):
        s_nom = lo_pad + ci * CHUNK
        s_c = jnp.minimum(s_nom, N_EDGES_K - CHUNK)
        q = s_c // SUB
        m_lo = jnp.maximum(lo, s_nom)
        m_hi = jnp.minimum(hi, s_nom + CHUNK)

        pltpu.sync_copy(cols_hbm.at[pl.ds(q, NSUBC)], colsv)
        pltpu.sync_copy(rows_hbm.at[pl.ds(q, NSUBC)], rowsv)
        pltpu.sync_copy(vals_hbm.at[pl.ds(q, NSUBC)], valsv)

        descs = [
            pltpu.async_copy(ego_hbm.at[colsv.at[j]],
                             gath.at[pl.ds(j * SUB, SUB)], sem)
            for j in range(NSUBC)
        ]
        for d in descs:
            d.wait()

        for j in range(NSUBC):
            for kk in range(SUB // LANES):
                off = kk * LANES
                ev = s_c + j * SUB + off + iota
                vv = valsv[j, pl.ds(off, LANES)]
                rv = rowsv[j, pl.ds(off, LANES)]
                valid = (ev >= m_lo) & (ev < m_hi)
                valsv[j, pl.ds(off, LANES)] = jnp.where(valid, vv, 0.0)
                lr = jnp.minimum(jnp.maximum(rv - base, 0), ROWS_W - 1)
                lrowsv[j, pl.ds(off, LANES)] = lr + acc_lo

        def mul_body(i, _):
            for u in range(4):
                e = i * 4 + u
                v = valsv[e >> 7, e & (SUB - 1)]
                gath[e, pl.ds(0, LANES)] = gath[e, pl.ds(0, LANES)] * v
                gath[e, pl.ds(LANES, LANES)] = gath[e, pl.ds(LANES, LANES)] * v
            return 0

        lax.fori_loop(0, CHUNK // 4, mul_body, 0)

        adescs = [
            pltpu.async_copy(gath.at[pl.ds(j * SUB, SUB)],
                             acc_sh.at[lrowsv.at[j]], sem, add=True)
            for j in range(NSUBC)
        ]
        for d in adescs:
            d.wait()
        return 0

    lax.fori_loop(0, nchunks, chunk_body, 0)

    pltpu.sync_copy(acc_sh.at[pl.ds(acc_lo, ROWS_W)],
                    out_hbm.at[pl.ds(base, ROWS_W)])


@jax.jit
def _spmm(ego, cols2d, rows2d, vals2d, bounds):
    mesh = plsc.VectorSubcoreMesh(core_axis_name="c", subcore_axis_name="s")
    kern = pl.kernel(
        _spmm_body,
        out_type=jax.ShapeDtypeStruct((N_NODES_K, EMB_K), jnp.float32),
        mesh=mesh,
        scratch_types=[
            pltpu.VMEM((NW + 8,), jnp.int32),
            pltpu.VMEM((NSUBC, SUB), jnp.int32),
            pltpu.VMEM((NSUBC, SUB), jnp.int32),
            pltpu.VMEM((NSUBC, SUB), jnp.float32),
            pltpu.VMEM((NSUBC, SUB), jnp.int32),
            pltpu.VMEM((CHUNK, EMB_K), jnp.float32),
            pltpu.VMEM_SHARED((ROWS_CORE, EMB_K), jnp.float32),
            pltpu.SemaphoreType.DMA,
        ],
    )
    return kern(ego, cols2d, rows2d, vals2d, bounds)


def kernel(user_emb, item_emb, adj_vals, adj_rows, adj_cols):
    ego = jnp.concatenate([user_emb, item_emb], axis=0)
    edges = jnp.arange(0, N_NODES_K + 1, ROWS_W, dtype=jnp.int32)
    bounds = jnp.searchsorted(adj_rows, edges, side="left").astype(jnp.int32)
    bounds = jnp.pad(bounds, (0, NW + 8 - bounds.shape[0]))
    cols2d = adj_cols.reshape(N_EDGES_K // SUB, SUB)
    rows2d = adj_rows.reshape(N_EDGES_K // SUB, SUB)
    vals2d = adj_vals.reshape(N_EDGES_K // SUB, SUB)
    mean_item_emb = _spmm(ego, cols2d, rows2d, vals2d, bounds)
    return (user_emb, item_emb, mean_item_emb)


# speedup vs baseline: 22.2117x; 22.2117x over previous
import jax
import jax.numpy as jnp
from jax import lax
from jax.experimental import pallas as pl
from jax.experimental.pallas import tpu as pltpu
from jax.experimental.pallas import tpu_sc as plsc

N_USERS_K = 50000
N_ITEMS_K = 50000
N_NODES_K = N_USERS_K + N_ITEMS_K
N_EDGES_K = 1600000
EMB_K = 32

NCORES = 2
NSUB = 16
NW = NCORES * NSUB
ROWS_W = 3128
N_PAD = ROWS_W * NW
ROWS_CORE = ROWS_W * NSUB
CHUNK = 256
SUB = 128
NSUBC = CHUNK // SUB
LANES = 16
GROUPS = CHUNK // LANES


def _spmm_body(ego_hbm, cols_hbm, rows_hbm, vals_hbm, bounds_hbm, out_hbm,
               boundsv,
               cols0, rows0, vals0, lrows0, gath0, esem0, gsem0, ssem0,
               cols1, rows1, vals1, lrows1, gath1, esem1, gsem1, ssem1,
               acc_sh):
    c = lax.axis_index("c")
    s = lax.axis_index("s")
    wid = c * NSUB + s
    base = wid * ROWS_W
    acc_lo = s * ROWS_W

    b0 = (cols0, rows0, vals0, lrows0, gath0, esem0, gsem0, ssem0)
    b1 = (cols1, rows1, vals1, lrows1, gath1, esem1, gsem1, ssem1)

    pltpu.sync_copy(bounds_hbm, boundsv)
    bv = boundsv[wid, pl.ds(0, LANES)]
    lo = bv[0]
    hi = bv[1]
    lo_pad = (lo // CHUNK) * CHUNK
    nchunks = (hi - lo_pad + CHUNK - 1) // CHUNK
    npairs = jnp.maximum((nchunks + 1) // 2, 1)
    iota = lax.iota(jnp.int32, LANES)

    def chunk_start(ci):
        s_nom = lo_pad + ci * CHUNK
        return jnp.minimum(s_nom, N_EDGES_K - CHUNK), s_nom

    def fire_edges(bufs, ci):
        colsv, rowsv, valsv = bufs[0], bufs[1], bufs[2]
        s_c, _ = chunk_start(ci)
        pltpu.async_copy(cols_hbm.at[pl.ds(s_c, CHUNK)], colsv, bufs[5])
        pltpu.async_copy(rows_hbm.at[pl.ds(s_c, CHUNK)], rowsv, bufs[5])
        pltpu.async_copy(vals_hbm.at[pl.ds(s_c, CHUNK)], valsv, bufs[5])

    def wait_edges(bufs):
        pltpu.make_async_copy(cols_hbm.at[pl.ds(0, CHUNK)], bufs[0], bufs[5]).wait()
        pltpu.make_async_copy(rows_hbm.at[pl.ds(0, CHUNK)], bufs[1], bufs[5]).wait()
        pltpu.make_async_copy(vals_hbm.at[pl.ds(0, CHUNK)], bufs[2], bufs[5]).wait()

    def fire_gathers(bufs):
        for j in range(NSUBC):
            pltpu.async_copy(ego_hbm.at[bufs[0].at[pl.ds(j * SUB, SUB)]],
                             bufs[4].at[pl.ds(j * SUB, SUB)], bufs[6])

    def wait_gathers(bufs):
        for j in range(NSUBC):
            pltpu.make_async_copy(ego_hbm.at[bufs[0].at[pl.ds(j * SUB, SUB)]],
                                  bufs[4].at[pl.ds(j * SUB, SUB)], bufs[6]).wait()

    def fire_scatter(bufs):
        for j in range(NSUBC):
            pltpu.async_copy(bufs[4].at[pl.ds(j * SUB, SUB)],
                             acc_sh.at[bufs[3].at[j]], bufs[7], add=True)

    def wait_scatter(bufs):
        for j in range(NSUBC):
            pltpu.make_async_copy(bufs[4].at[pl.ds(j * SUB, SUB)],
                                  acc_sh.at[bufs[3].at[j]], bufs[7]).wait()

    def maskpass(bufs, ci):
        s_c, s_nom = chunk_start(ci)
        m_lo = jnp.maximum(lo, s_nom)
        m_hi = jnp.minimum(hi, s_nom + CHUNK)
        rowsv, valsv, lrowsv = bufs[1], bufs[2], bufs[3]
        for g in range(GROUPS):
            off = g * LANES
            ev = s_c + off + iota
            vv = valsv[pl.ds(off, LANES)]
            rv = rowsv[pl.ds(off, LANES)]
            valid = (ev >= m_lo) & (ev < m_hi)
            valsv[pl.ds(off, LANES)] = jnp.where(valid, vv, 0.0)
            lr = jnp.minimum(jnp.maximum(rv - base, 0), ROWS_W - 1)
            lrowsv[g // (SUB // LANES),
                   pl.ds((g % (SUB // LANES)) * LANES, LANES)] = lr + acc_lo

    def mulpass(bufs):
        valsv, gath = bufs[2], bufs[4]

        def mul_body(i, _):
            vv = valsv[pl.ds(i * LANES, LANES)]
            e0 = i * LANES
            for u in range(LANES):
                e = e0 + u
                v = vv[u]
                gath[e, pl.ds(0, LANES)] = gath[e, pl.ds(0, LANES)] * v
                gath[e, pl.ds(LANES, LANES)] = gath[e, pl.ds(LANES, LANES)] * v
            return 0

        lax.fori_loop(0, GROUPS, mul_body, 0)

    zvec = jnp.zeros((LANES,), jnp.float32)

    def zero_body(i, _):
        gath0[i >> 1, pl.ds((i & 1) * LANES, LANES)] = zvec
        return 0

    lax.fori_loop(0, CHUNK * 2, zero_body, 0)
    for k in range(ROWS_W // CHUNK):
        pltpu.sync_copy(gath0, acc_sh.at[pl.ds(acc_lo + k * CHUNK, CHUNK)])
    rem = ROWS_W % CHUNK
    if rem:
        pltpu.sync_copy(
            gath0.at[pl.ds(0, rem)],
            acc_sh.at[pl.ds(acc_lo + (ROWS_W // CHUNK) * CHUNK, rem)])

    fire_edges(b0, 0)
    fire_edges(b1, 1)
    wait_edges(b0)
    fire_gathers(b0)
    maskpass(b0, 0)

    def half(cur, nxt, ci, skip_scatter_wait=None):
        wait_edges(nxt)
        if skip_scatter_wait is None:
            wait_scatter(nxt)
        else:
            @pl.when(jnp.logical_not(skip_scatter_wait))
            def _():
                wait_scatter(nxt)
        fire_gathers(nxt)
        maskpass(nxt, ci + 1)
        wait_gathers(cur)
        mulpass(cur)
        fire_scatter(cur)
        fire_edges(cur, ci + 2)

    def pair_body(p, _):
        ci0 = 2 * p
        half(b0, b1, ci0, skip_scatter_wait=(p == 0))
        half(b1, b0, ci0 + 1)
        return 0

    lax.fori_loop(0, npairs, pair_body, 0)

    wait_gathers(b0)
    wait_edges(b1)
    wait_scatter(b1)

    pltpu.sync_copy(acc_sh.at[pl.ds(acc_lo, ROWS_W)],
                    out_hbm.at[pl.ds(base, ROWS_W)])


@jax.jit
def _spmm(ego, cols, rows, vals, bounds):
    mesh = plsc.VectorSubcoreMesh(core_axis_name="c", subcore_axis_name="s")
    dbuf = [
        pltpu.VMEM((CHUNK,), jnp.int32),
        pltpu.VMEM((CHUNK,), jnp.int32),
        pltpu.VMEM((CHUNK,), jnp.float32),
        pltpu.VMEM((NSUBC, SUB), jnp.int32),
        pltpu.VMEM((CHUNK, EMB_K), jnp.float32),
        pltpu.SemaphoreType.DMA,
        pltpu.SemaphoreType.DMA,
        pltpu.SemaphoreType.DMA,
    ]
    kern = pl.kernel(
        _spmm_body,
        out_type=jax.ShapeDtypeStruct((N_PAD, EMB_K), jnp.float32),
        mesh=mesh,
        scratch_types=(
            [pltpu.VMEM((NW, LANES), jnp.int32)] + dbuf + dbuf
            + [pltpu.VMEM_SHARED((ROWS_CORE, EMB_K), jnp.float32)]
        ),
        compiler_params=pltpu.CompilerParams(use_tc_tiling_on_sc=False),
    )
    return kern(ego, cols, rows, vals, bounds)


def kernel(user_emb, item_emb, adj_vals, adj_rows, adj_cols):
    ego = jnp.concatenate([user_emb, item_emb], axis=0)
    edges = jnp.minimum(jnp.arange(NW + 1, dtype=jnp.int32) * ROWS_W,
                        N_NODES_K)
    b = jnp.searchsorted(adj_rows, edges, side="left").astype(jnp.int32)
    bounds = jnp.zeros((NW, LANES), jnp.int32)
    bounds = bounds.at[:, 0].set(b[:-1]).at[:, 1].set(b[1:])
    out = _spmm(ego, adj_cols, adj_rows, adj_vals, bounds)
    return (user_emb, item_emb, out[:N_NODES_K])


# speedup vs baseline: 22.2750x; 1.0029x over previous
import jax
import jax.numpy as jnp
from jax import lax
from jax.experimental import pallas as pl
from jax.experimental.pallas import tpu as pltpu
from jax.experimental.pallas import tpu_sc as plsc

N_USERS_K = 50000
N_ITEMS_K = 50000
N_NODES_K = N_USERS_K + N_ITEMS_K
N_EDGES_K = 1600000
EMB_K = 32

NCORES = 2
NSUB = 16
NW = NCORES * NSUB
NPASS = 2
NWIN = NW * NPASS
WIN_BIG = 1568
WIN_SMALL = 1560
NBIG = 20
ROWS_CORE = WIN_BIG * NSUB
CHUNK = 512
SUB = 128
NSUBC = CHUNK // SUB
LANES = 16
GROUPS = CHUNK // LANES


def _spmm_body(ego_hbm, cols_hbm, rows_hbm, vals_hbm, bounds_hbm, out_hbm,
               boundsv,
               cols0, rows0, vals0, lrows0, gath0, esem0, gsem0, ssem0,
               cols1, rows1, vals1, lrows1, gath1, esem1, gsem1, ssem1,
               cols2, rows2, vals2, lrows2, gath2, esem2, gsem2, ssem2,
               acc_sh):
    c = lax.axis_index("c")
    s = lax.axis_index("s")
    wid = c * NSUB + s
    acc_lo = s * WIN_BIG

    s0 = (cols0, rows0, vals0, lrows0, gath0, esem0, gsem0, ssem0)
    s1 = (cols1, rows1, vals1, lrows1, gath1, esem1, gsem1, ssem1)
    s2 = (cols2, rows2, vals2, lrows2, gath2, esem2, gsem2, ssem2)

    pltpu.sync_copy(bounds_hbm, boundsv)
    iota = lax.iota(jnp.int32, LANES)
    zvec = jnp.zeros((LANES,), jnp.float32)

    def fire_edges(bufs, s_c):
        pltpu.async_copy(cols_hbm.at[pl.ds(s_c, CHUNK)], bufs[0], bufs[5])
        pltpu.async_copy(rows_hbm.at[pl.ds(s_c, CHUNK)], bufs[1], bufs[5])
        pltpu.async_copy(vals_hbm.at[pl.ds(s_c, CHUNK)], bufs[2], bufs[5])

    def wait_edges(bufs):
        pltpu.make_async_copy(cols_hbm.at[pl.ds(0, CHUNK)], bufs[0], bufs[5]).wait()
        pltpu.make_async_copy(rows_hbm.at[pl.ds(0, CHUNK)], bufs[1], bufs[5]).wait()
        pltpu.make_async_copy(vals_hbm.at[pl.ds(0, CHUNK)], bufs[2], bufs[5]).wait()

    def fire_gathers(bufs):
        for j in range(NSUBC):
            pltpu.async_copy(ego_hbm.at[bufs[0].at[pl.ds(j * SUB, SUB)]],
                             bufs[4].at[pl.ds(j * SUB, SUB)], bufs[6])

    def wait_gathers(bufs):
        for j in range(NSUBC):
            pltpu.make_async_copy(ego_hbm.at[bufs[0].at[pl.ds(j * SUB, SUB)]],
                                  bufs[4].at[pl.ds(j * SUB, SUB)], bufs[6]).wait()

    def fire_scatter(bufs):
        for j in range(NSUBC):
            pltpu.async_copy(bufs[4].at[pl.ds(j * SUB, SUB)],
                             acc_sh.at[bufs[3].at[j]], bufs[7], add=True)

    def wait_scatter(bufs):
        for j in range(NSUBC):
            pltpu.make_async_copy(bufs[4].at[pl.ds(j * SUB, SUB)],
                                  acc_sh.at[bufs[3].at[j]], bufs[7]).wait()

    def mulpass(bufs):
        valsv, gath = bufs[2], bufs[4]

        def mul_body(i, _):
            vv = valsv[pl.ds(i * LANES, LANES)]
            e0 = i * LANES
            for u in range(LANES):
                e = e0 + u
                v = vv[u]
                gath[e, pl.ds(0, LANES)] = gath[e, pl.ds(0, LANES)] * v
                gath[e, pl.ds(LANES, LANES)] = gath[e, pl.ds(LANES, LANES)] * v
            return 0

        lax.fori_loop(0, GROUPS, mul_body, 0)

    def run_pass(p):
        g = p * NW + wid
        base = WIN_BIG * g - 8 * jnp.maximum(g - NBIG, 0)
        sz = jnp.where(g < NBIG, WIN_BIG, WIN_SMALL)
        bv = boundsv[g, pl.ds(0, LANES)]
        lo = bv[0]
        hi = bv[1]
        lo_pad = (lo // CHUNK) * CHUNK
        nchunks = (hi - lo_pad + CHUNK - 1) // CHUNK
        ntrips = jnp.maximum((nchunks + 2) // 3, 1)

        def chunk_start(ci):
            s_nom = lo_pad + ci * CHUNK
            return jnp.minimum(s_nom, N_EDGES_K - CHUNK), s_nom

        def maskpass(bufs, ci):
            s_c, s_nom = chunk_start(ci)
            m_lo = jnp.maximum(lo, s_nom)
            m_hi = jnp.minimum(hi, s_nom + CHUNK)
            rowsv, valsv, lrowsv = bufs[1], bufs[2], bufs[3]

            def mask_body(gr, _):
                off = gr * LANES
                ev = s_c + off + iota
                vv = valsv[pl.ds(off, LANES)]
                rv = rowsv[pl.ds(off, LANES)]
                valid = (ev >= m_lo) & (ev < m_hi)
                valsv[pl.ds(off, LANES)] = jnp.where(valid, vv, 0.0)
                lr = jnp.minimum(jnp.maximum(rv - base, 0), sz - 1)
                lrowsv[gr // (SUB // LANES),
                       pl.ds((gr % (SUB // LANES)) * LANES, LANES)] = lr + acc_lo
                return 0

            lax.fori_loop(0, GROUPS, mask_body, 0)

        def zero_body(i, _):
            gath0[i >> 1, pl.ds((i & 1) * LANES, LANES)] = zvec
            return 0

        lax.fori_loop(0, CHUNK * 2, zero_body, 0)
        for k in range(WIN_BIG // CHUNK):
            pltpu.sync_copy(gath0, acc_sh.at[pl.ds(acc_lo + k * CHUNK, CHUNK)])
        rem = WIN_BIG % CHUNK
        if rem:
            pltpu.sync_copy(
                gath0.at[pl.ds(0, rem)],
                acc_sh.at[pl.ds(acc_lo + (WIN_BIG // CHUNK) * CHUNK, rem)])

        fire_edges(s0, chunk_start(0)[0])
        fire_edges(s1, chunk_start(1)[0])
        wait_edges(s0)
        fire_gathers(s0)
        maskpass(s0, 0)

        def step(cur, nxt, prv, ci, guard):
            wait_edges(nxt)
            if guard is None:
                wait_scatter(nxt)
            else:
                @pl.when(guard)
                def _():
                    wait_scatter(nxt)
            fire_gathers(nxt)
            maskpass(nxt, ci + 1)
            wait_gathers(cur)
            mulpass(cur)
            fire_scatter(cur)
            fire_edges(prv, chunk_start(ci + 2)[0])

        def trip_body(t, _):
            ci = 3 * t
            step(s0, s1, s2, ci, t > 0)
            step(s1, s2, s0, ci + 1, t > 0)
            step(s2, s0, s1, ci + 2, None)
            return 0

        lax.fori_loop(0, ntrips, trip_body, 0)

        wait_gathers(s0)
        wait_edges(s1)
        wait_scatter(s1)
        wait_scatter(s2)

        pltpu.sync_copy(acc_sh.at[pl.ds(acc_lo, WIN_SMALL)],
                        out_hbm.at[pl.ds(base, WIN_SMALL)])

        @pl.when(g < NBIG)
        def _():
            pltpu.sync_copy(
                acc_sh.at[pl.ds(acc_lo + WIN_SMALL, WIN_BIG - WIN_SMALL)],
                out_hbm.at[pl.ds(base + WIN_SMALL, WIN_BIG - WIN_SMALL)])

    for p in range(NPASS):
        run_pass(p)


@jax.jit
def _spmm(ego, cols, rows, vals, bounds):
    mesh = plsc.VectorSubcoreMesh(core_axis_name="c", subcore_axis_name="s")
    ring = []
    for _ in range(3):
        ring += [
            pltpu.VMEM((CHUNK,), jnp.int32),
            pltpu.VMEM((CHUNK,), jnp.int32),
            pltpu.VMEM((CHUNK,), jnp.float32),
            pltpu.VMEM((NSUBC, SUB), jnp.int32),
            pltpu.VMEM((CHUNK, EMB_K), jnp.float32),
            pltpu.SemaphoreType.DMA,
            pltpu.SemaphoreType.DMA,
            pltpu.SemaphoreType.DMA,
        ]
    kern = pl.kernel(
        _spmm_body,
        out_type=jax.ShapeDtypeStruct((N_NODES_K, EMB_K), jnp.float32),
        mesh=mesh,
        scratch_types=(
            [pltpu.VMEM((NWIN, LANES), jnp.int32)] + ring
            + [pltpu.VMEM_SHARED((ROWS_CORE, EMB_K), jnp.float32)]
        ),
        compiler_params=pltpu.CompilerParams(use_tc_tiling_on_sc=False),
    )
    return kern(ego, cols, rows, vals, bounds)


def kernel(user_emb, item_emb, adj_vals, adj_rows, adj_cols):
    ego = jnp.concatenate([user_emb, item_emb], axis=0)
    gg = jnp.arange(NWIN + 1, dtype=jnp.int32)
    edges = WIN_BIG * gg - 8 * jnp.maximum(gg - NBIG, 0)
    b = jnp.searchsorted(adj_rows, edges, side="left").astype(jnp.int32)
    bounds = jnp.zeros((NWIN, LANES), jnp.int32)
    bounds = bounds.at[:, 0].set(b[:-1]).at[:, 1].set(b[1:])
    out = _spmm(ego, adj_cols, adj_rows, adj_vals, bounds)
    return (user_emb, item_emb, out)


# speedup vs baseline: 24.3905x; 1.0950x over previous
import jax
import jax.numpy as jnp
from jax import lax
from jax.experimental import pallas as pl
from jax.experimental.pallas import tpu as pltpu
from jax.experimental.pallas import tpu_sc as plsc

N_USERS_K = 50000
N_ITEMS_K = 50000
N_NODES_K = N_USERS_K + N_ITEMS_K
N_EDGES_K = 1600000
EMB_K = 32

NCORES = 2
NSUB = 16
NW = NCORES * NSUB
NPASS = 2
NWIN = NW * NPASS
WIN_BIG = 1568
WIN_SMALL = 1560
NBIG = 20
ROWS_CORE = WIN_BIG * NSUB
CHUNK = 512
SUB = 128
NSUBC = CHUNK // SUB
LANES = 16
GROUPS = CHUNK // LANES


def _spmm_body(ego_hbm, cols_hbm, rows_hbm, vals_hbm, bounds_hbm, out_hbm,
               boundsv,
               cols0, rows0, vals0, lrows0, gath0, esem0, gsem0, ssem0,
               cols1, rows1, vals1, lrows1, gath1, esem1, gsem1, ssem1,
               cols2, rows2, vals2, lrows2, gath2, esem2, gsem2, ssem2,
               acc_sh):
    c = lax.axis_index("c")
    s = lax.axis_index("s")
    wid = c * NSUB + s
    acc_lo = s * WIN_BIG

    s0 = (cols0, rows0, vals0, lrows0, gath0, esem0, gsem0, ssem0)
    s1 = (cols1, rows1, vals1, lrows1, gath1, esem1, gsem1, ssem1)
    s2 = (cols2, rows2, vals2, lrows2, gath2, esem2, gsem2, ssem2)

    pltpu.sync_copy(bounds_hbm, boundsv)
    iota = lax.iota(jnp.int32, LANES)
    zvec = jnp.zeros((LANES,), jnp.float32)

    def fire_edges(bufs, s_c):
        pltpu.async_copy(cols_hbm.at[pl.ds(s_c, CHUNK)], bufs[0], bufs[5])
        pltpu.async_copy(rows_hbm.at[pl.ds(s_c, CHUNK)], bufs[1], bufs[5])
        pltpu.async_copy(vals_hbm.at[pl.ds(s_c, CHUNK)], bufs[2], bufs[5])

    def wait_edges(bufs):
        pltpu.make_async_copy(cols_hbm.at[pl.ds(0, CHUNK)], bufs[0], bufs[5]).wait()
        pltpu.make_async_copy(rows_hbm.at[pl.ds(0, CHUNK)], bufs[1], bufs[5]).wait()
        pltpu.make_async_copy(vals_hbm.at[pl.ds(0, CHUNK)], bufs[2], bufs[5]).wait()

    def fire_gathers(bufs):
        for j in range(NSUBC):
            pltpu.async_copy(ego_hbm.at[bufs[0].at[pl.ds(j * SUB, SUB)]],
                             bufs[4].at[pl.ds(j * SUB, SUB)], bufs[6])

    def wait_gathers(bufs):
        for j in range(NSUBC):
            pltpu.make_async_copy(ego_hbm.at[bufs[0].at[pl.ds(j * SUB, SUB)]],
                                  bufs[4].at[pl.ds(j * SUB, SUB)], bufs[6]).wait()

    def fire_scatter(bufs):
        for j in range(NSUBC):
            pltpu.async_copy(bufs[4].at[pl.ds(j * SUB, SUB)],
                             acc_sh.at[bufs[3].at[j]], bufs[7], add=True)

    def wait_scatter(bufs):
        for j in range(NSUBC):
            pltpu.make_async_copy(bufs[4].at[pl.ds(j * SUB, SUB)],
                                  acc_sh.at[bufs[3].at[j]], bufs[7]).wait()

    def mulpass(bufs):
        valsv, gath = bufs[2], bufs[4]

        def mul_body(i, _):
            vv = valsv[pl.ds(i * LANES, LANES)]
            e0 = i * LANES
            for u in range(LANES):
                e = e0 + u
                v = vv[u]
                gath[e, pl.ds(0, LANES)] = gath[e, pl.ds(0, LANES)] * v
                gath[e, pl.ds(LANES, LANES)] = gath[e, pl.ds(LANES, LANES)] * v
            return 0

        lax.fori_loop(0, GROUPS, mul_body, 0)

    def run_pass(p):
        g = p * NW + wid
        base = WIN_BIG * g - 8 * jnp.maximum(g - NBIG, 0)
        sz = jnp.where(g < NBIG, WIN_BIG, WIN_SMALL)
        bv = boundsv[g, pl.ds(0, LANES)]
        lo = (bv[0] // CHUNK) * CHUNK
        hi = bv[1]
        nchunks = (hi - lo + CHUNK - 1) // CHUNK
        ntrips = jnp.maximum((nchunks + 2) // 3, 1)

        def chunk_start(ci):
            return jnp.minimum(lo + ci * CHUNK, N_EDGES_K - CHUNK)

        def maskpass(bufs, ci):
            szz = jnp.where(ci < nchunks, sz, 0)
            rowsv, valsv, lrowsv = bufs[1], bufs[2], bufs[3]

            def mask_body(gr, _):
                off = gr * LANES
                vv = valsv[pl.ds(off, LANES)]
                rv = rowsv[pl.ds(off, LANES)]
                valid = (rv >= base) & (rv < base + szz)
                valsv[pl.ds(off, LANES)] = jnp.where(valid, vv, 0.0)
                lr = jnp.minimum(jnp.maximum(rv - base, 0), sz - 1)
                lrowsv[gr // (SUB // LANES),
                       pl.ds((gr % (SUB // LANES)) * LANES, LANES)] = lr + acc_lo
                return 0

            lax.fori_loop(0, GROUPS, mask_body, 0)

        def zero_body(i, _):
            gath0[i >> 1, pl.ds((i & 1) * LANES, LANES)] = zvec
            return 0

        lax.fori_loop(0, CHUNK * 2, zero_body, 0)
        for k in range(WIN_BIG // CHUNK):
            pltpu.sync_copy(gath0, acc_sh.at[pl.ds(acc_lo + k * CHUNK, CHUNK)])
        rem = WIN_BIG % CHUNK
        if rem:
            pltpu.sync_copy(
                gath0.at[pl.ds(0, rem)],
                acc_sh.at[pl.ds(acc_lo + (WIN_BIG // CHUNK) * CHUNK, rem)])

        fire_edges(s0, chunk_start(0))
        fire_edges(s1, chunk_start(1))
        wait_edges(s0)
        fire_gathers(s0)
        maskpass(s0, 0)

        def step(cur, nxt, prv, ci, guard):
            wait_edges(nxt)
            if guard is None:
                wait_scatter(nxt)
            else:
                @pl.when(guard)
                def _():
                    wait_scatter(nxt)
            fire_gathers(nxt)
            maskpass(nxt, ci + 1)
            wait_gathers(cur)
            mulpass(cur)
            fire_scatter(cur)
            fire_edges(prv, chunk_start(ci + 2))

        def trip_body(t, _):
            ci = 3 * t
            step(s0, s1, s2, ci, t > 0)
            step(s1, s2, s0, ci + 1, t > 0)
            step(s2, s0, s1, ci + 2, None)
            return 0

        lax.fori_loop(0, ntrips, trip_body, 0)

        wait_gathers(s0)
        wait_edges(s1)
        wait_scatter(s1)
        wait_scatter(s2)

        pltpu.sync_copy(acc_sh.at[pl.ds(acc_lo, WIN_SMALL)],
                        out_hbm.at[pl.ds(base, WIN_SMALL)])

        @pl.when(g < NBIG)
        def _():
            pltpu.sync_copy(
                acc_sh.at[pl.ds(acc_lo + WIN_SMALL, WIN_BIG - WIN_SMALL)],
                out_hbm.at[pl.ds(base + WIN_SMALL, WIN_BIG - WIN_SMALL)])

    for p in range(NPASS):
        run_pass(p)


@jax.jit
def _spmm(ego, cols, rows, vals, bounds):
    mesh = plsc.VectorSubcoreMesh(core_axis_name="c", subcore_axis_name="s")
    ring = []
    for _ in range(3):
        ring += [
            pltpu.VMEM((CHUNK,), jnp.int32),
            pltpu.VMEM((CHUNK,), jnp.int32),
            pltpu.VMEM((CHUNK,), jnp.float32),
            pltpu.VMEM((NSUBC, SUB), jnp.int32),
            pltpu.VMEM((CHUNK, EMB_K), jnp.float32),
            pltpu.SemaphoreType.DMA,
            pltpu.SemaphoreType.DMA,
            pltpu.SemaphoreType.DMA,
        ]
    kern = pl.kernel(
        _spmm_body,
        out_type=jax.ShapeDtypeStruct((N_NODES_K, EMB_K), jnp.float32),
        mesh=mesh,
        scratch_types=(
            [pltpu.VMEM((NWIN, LANES), jnp.int32)] + ring
            + [pltpu.VMEM_SHARED((ROWS_CORE, EMB_K), jnp.float32)]
        ),
        compiler_params=pltpu.CompilerParams(use_tc_tiling_on_sc=False),
    )
    return kern(ego, cols, rows, vals, bounds)


def kernel(user_emb, item_emb, adj_vals, adj_rows, adj_cols):
    ego = jnp.concatenate([user_emb, item_emb], axis=0)
    STRIDE = 1024
    gg = jnp.arange(NWIN + 1, dtype=jnp.int32)
    edges = WIN_BIG * gg - 8 * jnp.maximum(gg - NBIG, 0)
    rows_s = adj_rows[::STRIDE]
    c = jnp.sum((rows_s[None, :] < edges[:, None]).astype(jnp.int32), axis=1)
    lo_cov = jnp.maximum(c[:-1] - 1, 0) * STRIDE
    hi_cov = jnp.minimum((c[1:] + 1) * STRIDE, N_EDGES_K)
    bounds = jnp.zeros((NWIN, LANES), jnp.int32)
    bounds = bounds.at[:, 0].set(lo_cov).at[:, 1].set(hi_cov)
    out = _spmm(ego, adj_cols, adj_rows, adj_vals, bounds)
    return (user_emb, item_emb, out)


# speedup vs baseline: 27.4232x; 1.1243x over previous
import jax
import jax.numpy as jnp
from jax import lax
from jax.experimental import pallas as pl
from jax.experimental.pallas import tpu as pltpu
from jax.experimental.pallas import tpu_sc as plsc

N_USERS_K = 50000
N_ITEMS_K = 50000
N_NODES_K = N_USERS_K + N_ITEMS_K
N_EDGES_K = 1600000
EMB_K = 32

NCORES = 2
NSUB = 16
NW = NCORES * NSUB
NPASS = 2
NWIN = NW * NPASS
WIN_BIG = 1568
WIN_SMALL = 1560
NBIG = 20
ROWS_CORE = WIN_BIG * NSUB
CHUNK = 512
SUB = 128
NSUBC = CHUNK // SUB
LANES = 16
GROUPS = CHUNK // LANES


def _spmm_body(ego_hbm, cols_hbm, rows_hbm, vals_hbm, bounds_hbm, out_hbm,
               boundsv,
               cols0, rows0, vals0, lrows0, gath0, esem0, gsem0, ssem0,
               cols1, rows1, vals1, lrows1, gath1, esem1, gsem1, ssem1,
               cols2, rows2, vals2, lrows2, gath2, esem2, gsem2, ssem2,
               acc_sh):
    c = lax.axis_index("c")
    s = lax.axis_index("s")
    wid = c * NSUB + s
    acc_lo = s * WIN_BIG

    s0 = (cols0, rows0, vals0, lrows0, gath0, esem0, gsem0, ssem0)
    s1 = (cols1, rows1, vals1, lrows1, gath1, esem1, gsem1, ssem1)
    s2 = (cols2, rows2, vals2, lrows2, gath2, esem2, gsem2, ssem2)

    pltpu.sync_copy(bounds_hbm, boundsv)
    iota = lax.iota(jnp.int32, LANES)
    zvec = jnp.zeros((LANES,), jnp.float32)

    def fire_edges(bufs, s_c):
        pltpu.async_copy(cols_hbm.at[pl.ds(s_c, CHUNK)], bufs[0], bufs[5])
        pltpu.async_copy(rows_hbm.at[pl.ds(s_c, CHUNK)], bufs[1], bufs[5])
        pltpu.async_copy(vals_hbm.at[pl.ds(s_c, CHUNK)], bufs[2], bufs[5])

    def wait_edges(bufs):
        pltpu.make_async_copy(cols_hbm.at[pl.ds(0, CHUNK)], bufs[0], bufs[5]).wait()
        pltpu.make_async_copy(rows_hbm.at[pl.ds(0, CHUNK)], bufs[1], bufs[5]).wait()
        pltpu.make_async_copy(vals_hbm.at[pl.ds(0, CHUNK)], bufs[2], bufs[5]).wait()

    def fire_gathers(bufs):
        for j in range(NSUBC):
            pltpu.async_copy(ego_hbm.at[bufs[0].at[pl.ds(j * SUB, SUB)]],
                             bufs[4].at[pl.ds(j * SUB, SUB)], bufs[6])

    def wait_gathers(bufs):
        for j in range(NSUBC):
            pltpu.make_async_copy(ego_hbm.at[bufs[0].at[pl.ds(j * SUB, SUB)]],
                                  bufs[4].at[pl.ds(j * SUB, SUB)], bufs[6]).wait()

    def fire_scatter(bufs):
        for j in range(NSUBC):
            pltpu.async_copy(bufs[4].at[pl.ds(j * SUB, SUB)],
                             acc_sh.at[bufs[3].at[j]], bufs[7], add=True)

    def wait_scatter(bufs):
        for j in range(NSUBC):
            pltpu.make_async_copy(bufs[4].at[pl.ds(j * SUB, SUB)],
                                  acc_sh.at[bufs[3].at[j]], bufs[7]).wait()

    def mulpass(bufs):
        valsv, gath = bufs[2], bufs[4]

        def mul_body(i, _):
            vv = valsv[pl.ds(i * LANES, LANES)]
            e0 = i * LANES
            for u in range(LANES):
                e = e0 + u
                v = vv[u]
                gath[e, pl.ds(0, LANES)] = gath[e, pl.ds(0, LANES)] * v
                gath[e, pl.ds(LANES, LANES)] = gath[e, pl.ds(LANES, LANES)] * v
            return 0

        lax.fori_loop(0, GROUPS, mul_body, 0)

    def run_pass(p):
        g = p * NW + wid
        base = WIN_BIG * g - 8 * jnp.maximum(g - NBIG, 0)
        sz = jnp.where(g < NBIG, WIN_BIG, WIN_SMALL)
        bv = boundsv[g, pl.ds(0, LANES)]
        lo = (bv[0] // CHUNK) * CHUNK
        hi = bv[1]
        nchunks = (hi - lo + CHUNK - 1) // CHUNK
        ntrips = jnp.maximum((nchunks + 2) // 3, 1)

        def chunk_start(ci):
            return jnp.minimum(lo + ci * CHUNK, N_EDGES_K - CHUNK)

        def maskpass(bufs, ci):
            szz = jnp.where(ci < nchunks, sz, 0)
            rowsv, valsv, lrowsv = bufs[1], bufs[2], bufs[3]

            def mask_body(gr, _):
                off = gr * LANES
                vv = valsv[pl.ds(off, LANES)]
                rv = rowsv[pl.ds(off, LANES)]
                valid = (rv >= base) & (rv < base + szz)
                valsv[pl.ds(off, LANES)] = jnp.where(valid, vv, 0.0)
                lr = jnp.minimum(jnp.maximum(rv - base, 0), sz - 1)
                lrowsv[gr // (SUB // LANES),
                       pl.ds((gr % (SUB // LANES)) * LANES, LANES)] = lr + acc_lo
                return 0

            lax.fori_loop(0, GROUPS, mask_body, 0)

        def zero_body(i, _):
            gath0[i >> 1, pl.ds((i & 1) * LANES, LANES)] = zvec
            return 0

        lax.fori_loop(0, CHUNK * 2, zero_body, 0)
        for k in range(WIN_BIG // CHUNK):
            pltpu.sync_copy(gath0, acc_sh.at[pl.ds(acc_lo + k * CHUNK, CHUNK)])
        rem = WIN_BIG % CHUNK
        if rem:
            pltpu.sync_copy(
                gath0.at[pl.ds(0, rem)],
                acc_sh.at[pl.ds(acc_lo + (WIN_BIG // CHUNK) * CHUNK, rem)])

        fire_edges(s0, chunk_start(0))
        fire_edges(s1, chunk_start(1))
        wait_edges(s0)
        fire_gathers(s0)
        maskpass(s0, 0)

        def step(cur, nxt, prv, ci, guard):
            wait_edges(nxt)
            if guard is None:
                wait_scatter(nxt)
            else:
                @pl.when(guard)
                def _():
                    wait_scatter(nxt)
            fire_gathers(nxt)
            maskpass(nxt, ci + 1)
            wait_gathers(cur)
            mulpass(cur)
            fire_scatter(cur)
            fire_edges(prv, chunk_start(ci + 2))

        def trip_body(t, _):
            ci = 3 * t
            step(s0, s1, s2, ci, t > 0)
            step(s1, s2, s0, ci + 1, t > 0)
            step(s2, s0, s1, ci + 2, None)
            return 0

        lax.fori_loop(0, ntrips, trip_body, 0)

        wait_gathers(s0)
        wait_edges(s1)
        wait_scatter(s1)
        wait_scatter(s2)

        pltpu.sync_copy(acc_sh.at[pl.ds(acc_lo, WIN_SMALL)],
                        out_hbm.at[pl.ds(base, WIN_SMALL)])

        @pl.when(g < NBIG)
        def _():
            pltpu.sync_copy(
                acc_sh.at[pl.ds(acc_lo + WIN_SMALL, WIN_BIG - WIN_SMALL)],
                out_hbm.at[pl.ds(base + WIN_SMALL, WIN_BIG - WIN_SMALL)])

    for p in range(NPASS):
        run_pass(p)


@jax.jit
def _spmm(ego, cols, rows, vals, bounds):
    mesh = plsc.VectorSubcoreMesh(core_axis_name="c", subcore_axis_name="s")
    ring = []
    for _ in range(3):
        ring += [
            pltpu.VMEM((CHUNK,), jnp.int32),
            pltpu.VMEM((CHUNK,), jnp.int32),
            pltpu.VMEM((CHUNK,), jnp.float32),
            pltpu.VMEM((NSUBC, SUB), jnp.int32),
            pltpu.VMEM((CHUNK, EMB_K), jnp.float32),
            pltpu.SemaphoreType.DMA,
            pltpu.SemaphoreType.DMA,
            pltpu.SemaphoreType.DMA,
        ]
    kern = pl.kernel(
        _spmm_body,
        out_type=jax.ShapeDtypeStruct((N_NODES_K, EMB_K), jnp.float32),
        mesh=mesh,
        scratch_types=(
            [pltpu.VMEM((NWIN, LANES), jnp.int32)] + ring
            + [pltpu.VMEM_SHARED((ROWS_CORE, EMB_K), jnp.float32)]
        ),
        compiler_params=pltpu.CompilerParams(use_tc_tiling_on_sc=False),
    )
    return kern(ego, cols, rows, vals, bounds)


def kernel(user_emb, item_emb, adj_vals, adj_rows, adj_cols):
    ego = jnp.concatenate([user_emb, item_emb], axis=0)
    STRIDE = 128
    gg = jnp.arange(NWIN + 1, dtype=jnp.int32)
    edges = WIN_BIG * gg - 8 * jnp.maximum(gg - NBIG, 0)
    rows_s = adj_rows[::STRIDE]
    c = jnp.sum((rows_s[None, :] < edges[:, None]).astype(jnp.int32), axis=1)
    lo_cov = (jnp.maximum(c[:-1] - 1, 0) * STRIDE) // CHUNK * CHUNK
    hi_cov = jnp.minimum((c[1:] + 1) * STRIDE, N_EDGES_K)
    bounds = jnp.zeros((NWIN, LANES), jnp.int32)
    bounds = bounds.at[:, 0].set(lo_cov).at[:, 1].set(hi_cov)
    out = _spmm(ego, adj_cols, adj_rows, adj_vals, bounds)
    return (user_emb, item_emb, out)
